# Initial kernel scaffold; baseline (speedup 1.0000x reference)
import jax
import jax.numpy as jnp
from jax.experimental import pallas as pl


def kernel(x, values):
    raise NotImplementedError("write your pallas kernel here")


# speedup vs baseline: 3.8971x; 3.8971x over previous
import functools

import jax
import jax.numpy as jnp
from jax import lax
from jax.experimental import pallas as pl
from jax.experimental.pallas import tpu as pltpu
from jax.experimental.pallas import tpu_sc as plsc

_NC = 2
_NS = 16
_L = 16
_NW = _NC * _NS


def _vq_body(nk, per_w, x_hbm, v0_hbm, istep_hbm, vals_hbm,
             q_hbm, idx_hbm, x_v, q_v, idx_v, vals_v, v0_v, istep_v):
    wid = lax.axis_index("c") * _NS + lax.axis_index("s")
    base = wid * per_w
    pltpu.sync_copy(x_hbm.at[pl.ds(base, per_w)], x_v)
    pltpu.sync_copy(vals_hbm, vals_v)
    pltpu.sync_copy(v0_hbm, v0_v)
    pltpu.sync_copy(istep_hbm, istep_v)
    v0 = v0_v[...]
    istep = istep_v[...]

    def body(i, carry):
        s = pl.ds(i * _L, _L)
        xv = x_v[s]
        t = (xv - v0) * istep
        ki = jnp.clip(t.astype(jnp.int32), 0, nk - 2)
        k1 = ki + 1
        vk = plsc.load_gather(vals_v, [ki])
        vk1 = plsc.load_gather(vals_v, [k1])
        m = jnp.abs(xv - vk) <= jnp.abs(xv - vk1)
        q_v[s] = jnp.where(m, vk, vk1)
        idx_v[s] = jnp.where(m, ki, k1)
        return carry

    lax.fori_loop(0, per_w // _L, body, 0)
    pltpu.sync_copy(q_v, q_hbm.at[pl.ds(base, per_w)])
    pltpu.sync_copy(idx_v, idx_hbm.at[pl.ds(base, per_w)])


@functools.partial(jax.jit, static_argnums=(0, 1))
def _vq_call(n, nk, x, v0, istep, values):
    per_w = n // _NW
    mesh = plsc.VectorSubcoreMesh(core_axis_name="c", subcore_axis_name="s")
    return pl.kernel(
        functools.partial(_vq_body, nk, per_w),
        out_type=(
            jax.ShapeDtypeStruct((n,), jnp.float32),
            jax.ShapeDtypeStruct((n,), jnp.int32),
        ),
        mesh=mesh,
        compiler_params=pltpu.CompilerParams(needs_layout_passes=False),
        scratch_types=[
            pltpu.VMEM((per_w,), jnp.float32),
            pltpu.VMEM((per_w,), jnp.float32),
            pltpu.VMEM((per_w,), jnp.int32),
            pltpu.VMEM((nk,), jnp.float32),
            pltpu.VMEM((_L,), jnp.float32),
            pltpu.VMEM((_L,), jnp.float32),
        ],
    )(x, v0, istep, values)


def kernel(x, values):
    n = x.shape[0]
    nk = values.shape[0]
    v0 = jnp.broadcast_to(values[0], (_L,))
    istep = jnp.broadcast_to((nk - 1) / (values[-1] - values[0]), (_L,))
    q, idx = _vq_call(n, nk, x, v0, istep, values)
    return (x, q, q, idx)


# speedup vs baseline: 5.2791x; 1.3546x over previous
import functools

import jax
import jax.numpy as jnp
from jax import lax
from jax.experimental import pallas as pl
from jax.experimental.pallas import tpu as pltpu
from jax.experimental.pallas import tpu_sc as plsc

_NC = 2
_NS = 16
_L = 16
_NW = _NC * _NS
_CHUNKS = 8


def _vq_body(nk, per_w, x_hbm, v0_hbm, istep_hbm, vals_hbm,
             q_hbm, idx_hbm, x_v, q_v, idx_v, vals_v, v0_v, istep_v,
             sem_out, *sems_in):
    wid = lax.axis_index("c") * _NS + lax.axis_index("s")
    base = wid * per_w
    ch = per_w // _CHUNKS

    in_copies = []
    for c in range(_CHUNKS):
        in_copies.append(pltpu.async_copy(
            x_hbm.at[pl.ds(base + c * ch, ch)],
            x_v.at[pl.ds(c * ch, ch)],
            sems_in[c]))
    pltpu.sync_copy(vals_hbm, vals_v)
    pltpu.sync_copy(v0_hbm, v0_v)
    pltpu.sync_copy(istep_hbm, istep_v)
    v0 = v0_v[...]
    istep = istep_v[...]

    out_copies = []
    for c in range(_CHUNKS):
        in_copies[c].wait()

        @plsc.parallel_loop(c * (ch // _L), (c + 1) * (ch // _L), unroll=8)
        def body(i):
            s = pl.ds(i * _L, _L)
            xv = x_v[s]
            t = (xv - v0) * istep
            ki = jnp.clip(t.astype(jnp.int32), 0, nk - 2)
            k1 = ki + 1
            vk = plsc.load_gather(vals_v, [ki])
            vk1 = plsc.load_gather(vals_v, [k1])
            m = jnp.abs(xv - vk) <= jnp.abs(xv - vk1)
            q_v[s] = jnp.where(m, vk, vk1)
            idx_v[s] = jnp.where(m, ki, k1)

        out_copies.append(pltpu.async_copy(
            q_v.at[pl.ds(c * ch, ch)],
            q_hbm.at[pl.ds(base + c * ch, ch)],
            sem_out))
        out_copies.append(pltpu.async_copy(
            idx_v.at[pl.ds(c * ch, ch)],
            idx_hbm.at[pl.ds(base + c * ch, ch)],
            sem_out))
    for cp in out_copies:
        cp.wait()


@functools.partial(jax.jit, static_argnums=(0, 1))
def _vq_call(n, nk, x, v0, istep, values):
    per_w = n // _NW
    mesh = plsc.VectorSubcoreMesh(core_axis_name="c", subcore_axis_name="s")
    return pl.kernel(
        functools.partial(_vq_body, nk, per_w),
        out_type=(
            jax.ShapeDtypeStruct((n,), jnp.float32),
            jax.ShapeDtypeStruct((n,), jnp.int32),
        ),
        mesh=mesh,
        compiler_params=pltpu.CompilerParams(needs_layout_passes=False),
        scratch_types=[
            pltpu.VMEM((per_w,), jnp.float32),
            pltpu.VMEM((per_w,), jnp.float32),
            pltpu.VMEM((per_w,), jnp.int32),
            pltpu.VMEM((nk,), jnp.float32),
            pltpu.VMEM((_L,), jnp.float32),
            pltpu.VMEM((_L,), jnp.float32),
            pltpu.SemaphoreType.DMA,
        ] + [pltpu.SemaphoreType.DMA] * _CHUNKS,
    )(x, v0, istep, values)


def kernel(x, values):
    n = x.shape[0]
    nk = values.shape[0]
    v0 = jnp.broadcast_to(values[0], (_L,))
    istep = jnp.broadcast_to((nk - 1) / (values[-1] - values[0]), (_L,))
    q, idx = _vq_call(n, nk, x, v0, istep, values)
    return (x, q, q, idx)


# speedup vs baseline: 6.2811x; 1.1898x over previous
import functools

import jax
import jax.numpy as jnp
from jax import lax
from jax.experimental import pallas as pl
from jax.experimental.pallas import tpu as pltpu
from jax.experimental.pallas import tpu_sc as plsc

_NC = 2
_NS = 16
_L = 16
_NW = _NC * _NS
_CHUNKS = 8


def _vq_body(nk, per_w, x_hbm, vals_hbm,
             zc_hbm, q_hbm, qh_hbm, idx_hbm, x_v, q_v, idx_v, vals_v,
             sem_out, *sems_in):
    wid = lax.axis_index("c") * _NS + lax.axis_index("s")
    base = wid * per_w
    ch = per_w // _CHUNKS

    in_copies = []
    for c in range(_CHUNKS):
        in_copies.append(pltpu.async_copy(
            x_hbm.at[pl.ds(base + c * ch, ch)],
            x_v.at[pl.ds(c * ch, ch)],
            sems_in[c]))
    pltpu.sync_copy(vals_hbm, vals_v)

    vmin = jnp.full((_L,), jnp.inf, jnp.float32)
    vmax = jnp.full((_L,), -jnp.inf, jnp.float32)
    for j in range(nk // _L):
        vj = vals_v[pl.ds(j * _L, _L)]
        vmin = jnp.minimum(vmin, vj)
        vmax = jnp.maximum(vmax, vj)
    v0s = jnp.min(vmin)
    v63s = jnp.max(vmax)
    zf = jnp.zeros((_L,), jnp.float32)
    v0 = zf + v0s
    istep = (zf + jnp.float32(nk - 1)) / (zf + (v63s - v0s))

    out_copies = []
    for c in range(_CHUNKS):
        in_copies[c].wait()

        @plsc.parallel_loop(c * (ch // _L), (c + 1) * (ch // _L), unroll=8)
        def body(i):
            s = pl.ds(i * _L, _L)
            xv = x_v[s]
            t = (xv - v0) * istep
            ki = jnp.clip(t.astype(jnp.int32), 0, nk - 2)
            k1 = ki + 1
            vk = plsc.load_gather(vals_v, [ki])
            vk1 = plsc.load_gather(vals_v, [k1])
            m = jnp.abs(xv - vk) <= jnp.abs(xv - vk1)
            q_v[s] = jnp.where(m, vk, vk1)
            idx_v[s] = jnp.where(m, ki, k1)

        sl_v = pl.ds(c * ch, ch)
        sl_h = pl.ds(base + c * ch, ch)
        out_copies.append(pltpu.async_copy(q_v.at[sl_v], q_hbm.at[sl_h], sem_out))
        out_copies.append(pltpu.async_copy(q_v.at[sl_v], qh_hbm.at[sl_h], sem_out))
        out_copies.append(pltpu.async_copy(idx_v.at[sl_v], idx_hbm.at[sl_h], sem_out))
        out_copies.append(pltpu.async_copy(x_v.at[sl_v], zc_hbm.at[sl_h], sem_out))
    for cp in out_copies:
        cp.wait()


@functools.partial(jax.jit, static_argnums=(0, 1))
def _vq_call(n, nk, x, values):
    per_w = n // _NW
    mesh = plsc.VectorSubcoreMesh(core_axis_name="c", subcore_axis_name="s")
    return pl.kernel(
        functools.partial(_vq_body, nk, per_w),
        out_type=(
            jax.ShapeDtypeStruct((n,), jnp.float32),
            jax.ShapeDtypeStruct((n,), jnp.float32),
            jax.ShapeDtypeStruct((n,), jnp.float32),
            jax.ShapeDtypeStruct((n,), jnp.int32),
        ),
        mesh=mesh,
        compiler_params=pltpu.CompilerParams(needs_layout_passes=False),
        scratch_types=[
            pltpu.VMEM((per_w,), jnp.float32),
            pltpu.VMEM((per_w,), jnp.float32),
            pltpu.VMEM((per_w,), jnp.int32),
            pltpu.VMEM((nk,), jnp.float32),
            pltpu.SemaphoreType.DMA,
        ] + [pltpu.SemaphoreType.DMA] * _CHUNKS,
    )(x, values)


def kernel(x, values):
    n = x.shape[0]
    nk = values.shape[0]
    zc, q, qh, idx = _vq_call(n, nk, x, values)
    return (zc, q, qh, idx)


# speedup vs baseline: 6.5455x; 1.0421x over previous
import functools

import jax
import jax.numpy as jnp
from jax import lax
from jax.experimental import pallas as pl
from jax.experimental.pallas import tpu as pltpu
from jax.experimental.pallas import tpu_sc as plsc

_NC = 2
_NS = 16
_L = 16
_NW = _NC * _NS
_CHUNKS = 4


def _vq_body(nk, per_w, x_hbm, vals_hbm,
             zc_hbm, q_hbm, qh_hbm, idx_hbm, x_v, q_v, idx_v, vals_v,
             sem_out, *sems_in):
    wid = lax.axis_index("c") * _NS + lax.axis_index("s")
    base = wid * per_w
    ch = per_w // _CHUNKS

    in_copies = []
    for c in range(_CHUNKS):
        in_copies.append(pltpu.async_copy(
            x_hbm.at[pl.ds(base + c * ch, ch)],
            x_v.at[pl.ds(c * ch, ch)],
            sems_in[c]))
    pltpu.sync_copy(vals_hbm, vals_v)

    vmin = jnp.full((_L,), jnp.inf, jnp.float32)
    vmax = jnp.full((_L,), -jnp.inf, jnp.float32)
    for j in range(nk // _L):
        vj = vals_v[pl.ds(j * _L, _L)]
        vmin = jnp.minimum(vmin, vj)
        vmax = jnp.maximum(vmax, vj)
    v0s = jnp.min(vmin)
    v63s = jnp.max(vmax)
    zf = jnp.zeros((_L,), jnp.float32)
    v0 = zf + v0s
    istep = (zf + jnp.float32(nk - 1)) / (zf + (v63s - v0s))

    out_copies = []
    for c in range(_CHUNKS):
        in_copies[c].wait()

        @plsc.parallel_loop(c * (ch // _L), (c + 1) * (ch // _L), unroll=4)
        def body(i):
            s = pl.ds(i * _L, _L)
            xv = x_v[s]
            t = (xv - v0) * istep
            ki = jnp.clip(t.astype(jnp.int32), 0, nk - 2)
            k1 = ki + 1
            vk = plsc.load_gather(vals_v, [ki])
            vk1 = plsc.load_gather(vals_v, [k1])
            m = jnp.abs(xv - vk) <= jnp.abs(xv - vk1)
            q_v[s] = jnp.where(m, vk, vk1)
            idx_v[s] = jnp.where(m, ki, k1)

        sl_v = pl.ds(c * ch, ch)
        sl_h = pl.ds(base + c * ch, ch)
        out_copies.append(pltpu.async_copy(q_v.at[sl_v], q_hbm.at[sl_h], sem_out))
        out_copies.append(pltpu.async_copy(q_v.at[sl_v], qh_hbm.at[sl_h], sem_out))
        out_copies.append(pltpu.async_copy(idx_v.at[sl_v], idx_hbm.at[sl_h], sem_out))
        out_copies.append(pltpu.async_copy(x_v.at[sl_v], zc_hbm.at[sl_h], sem_out))
    for cp in out_copies:
        cp.wait()


@functools.partial(jax.jit, static_argnums=(0, 1))
def _vq_call(n, nk, x, values):
    per_w = n // _NW
    mesh = plsc.VectorSubcoreMesh(core_axis_name="c", subcore_axis_name="s")
    return pl.kernel(
        functools.partial(_vq_body, nk, per_w),
        out_type=(
            jax.ShapeDtypeStruct((n,), jnp.float32),
            jax.ShapeDtypeStruct((n,), jnp.float32),
            jax.ShapeDtypeStruct((n,), jnp.float32),
            jax.ShapeDtypeStruct((n,), jnp.int32),
        ),
        mesh=mesh,
        compiler_params=pltpu.CompilerParams(needs_layout_passes=False),
        scratch_types=[
            pltpu.VMEM((per_w,), jnp.float32),
            pltpu.VMEM((per_w,), jnp.float32),
            pltpu.VMEM((per_w,), jnp.int32),
            pltpu.VMEM((nk,), jnp.float32),
            pltpu.SemaphoreType.DMA,
        ] + [pltpu.SemaphoreType.DMA] * _CHUNKS,
    )(x, values)


def kernel(x, values):
    n = x.shape[0]
    nk = values.shape[0]
    zc, q, qh, idx = _vq_call(n, nk, x, values)
    return (zc, q, qh, idx)
